# Initial kernel scaffold; baseline (speedup 1.0000x reference)
import jax
import jax.numpy as jnp
from jax.experimental import pallas as pl


def kernel(x, embedding_list):
    raise NotImplementedError("write your pallas kernel here")


# speedup vs baseline: 3.7091x; 3.7091x over previous
import functools

import jax
import jax.numpy as jnp
from jax import lax
from jax.experimental import pallas as pl
from jax.experimental.pallas import tpu as pltpu
from jax.experimental.pallas import tpu_sc as plsc


def kernel(x, embedding_list):
    B = x.shape[0]
    V, D = embedding_list.shape
    info = plsc.get_sparse_core_info()
    NC, NS = info.num_cores, info.num_subcores
    NW = NC * NS
    CH = 128
    n_rows = B // CH
    rows_per_w = n_rows // NW
    x2 = x.reshape(n_rows, CH)

    mesh = plsc.VectorSubcoreMesh(core_axis_name="c", subcore_axis_name="s")

    @functools.partial(
        pl.kernel,
        out_type=jax.ShapeDtypeStruct((B, D), jnp.float32),
        mesh=mesh,
        scratch_types=[
            pltpu.VMEM((CH,), jnp.int32),
            pltpu.VMEM((CH, D), jnp.float32),
            pltpu.SemaphoreType.DMA,
        ],
    )
    def emb_kernel(x_hbm, tab_hbm, out_hbm, idx_v, rows_v, sem):
        wid = lax.axis_index("s") * NC + lax.axis_index("c")
        row0 = wid * rows_per_w

        def body(r, carry):
            pltpu.sync_copy(x_hbm.at[row0 + r], idx_v)
            pltpu.async_copy(tab_hbm.at[idx_v], rows_v, sem).wait()
            pltpu.sync_copy(rows_v, out_hbm.at[pl.ds((row0 + r) * CH, CH)])
            return carry

        lax.fori_loop(0, rows_per_w, body, 0)

    return emb_kernel(x2, embedding_list)


# speedup vs baseline: 3.9225x; 1.0575x over previous
import functools

import jax
import jax.numpy as jnp
from jax import lax
from jax.experimental import pallas as pl
from jax.experimental.pallas import tpu as pltpu
from jax.experimental.pallas import tpu_sc as plsc


def kernel(x, embedding_list):
    B = x.shape[0]
    V, D = embedding_list.shape
    info = plsc.get_sparse_core_info()
    NC, NS = info.num_cores, info.num_subcores
    NW = NC * NS
    CH = 128
    NB = 4
    LA = 2
    n_rows = B // CH
    rows_per_w = n_rows // NW
    x2 = x.reshape(n_rows, CH)

    mesh = plsc.VectorSubcoreMesh(core_axis_name="c", subcore_axis_name="s")

    @functools.partial(
        pl.kernel,
        out_type=jax.ShapeDtypeStruct((B, D), jnp.float32),
        mesh=mesh,
        scratch_types=[
            pltpu.VMEM((rows_per_w, CH), jnp.int32),
            pltpu.VMEM((NB, CH, D), jnp.float32),
            pltpu.SemaphoreType.DMA((NB,)),
            pltpu.SemaphoreType.DMA((NB,)),
        ],
    )
    def emb_kernel(x_hbm, tab_hbm, out_hbm, idx_v, rows_v, gsem, wsem):
        wid = lax.axis_index("s") * NC + lax.axis_index("c")
        row0 = wid * rows_per_w

        pltpu.sync_copy(x_hbm.at[pl.ds(row0, rows_per_w)], idx_v)

        def gather(g, s):
            return pltpu.make_async_copy(
                tab_hbm.at[idx_v.at[g]], rows_v.at[s], gsem.at[s]
            )

        def write(g, s):
            return pltpu.make_async_copy(
                rows_v.at[s], out_hbm.at[pl.ds((row0 + g) * CH, CH)], wsem.at[s]
            )

        for k in range(LA):
            gather(k, k).start()

        def body(i, carry):
            for s in range(NB):
                g = NB * i + s
                gn = g + LA
                sn = (s + LA) % NB

                @pl.when(gn < rows_per_w)
                def _():
                    @pl.when(g >= NB - LA)
                    def _():
                        write(g, sn).wait()

                    gather(gn, sn).start()

                gather(g, s).wait()
                write(g, s).start()
            return carry

        lax.fori_loop(0, rows_per_w // NB, body, 0)
        for s in range(NB):
            write(0, s).wait()

    return emb_kernel(x2, embedding_list)


# speedup vs baseline: 20.0709x; 5.1168x over previous
import functools

import jax
import jax.numpy as jnp
from jax import lax
from jax.experimental import pallas as pl
from jax.experimental.pallas import tpu as pltpu
from jax.experimental.pallas import tpu_sc as plsc


def kernel(x, embedding_list):
    B = x.shape[0]
    V, D = embedding_list.shape
    info = plsc.get_sparse_core_info()
    NC, NS = info.num_cores, info.num_subcores
    NW = NC * NS
    CH = 128
    NB = 4
    LA = 2
    n_rows = B // CH
    rows_per_w = n_rows // NW
    x2 = x.reshape(n_rows, CH)

    mesh = plsc.VectorSubcoreMesh(core_axis_name="c", subcore_axis_name="s")

    @functools.partial(
        pl.kernel,
        out_type=jax.ShapeDtypeStruct((B, D), jnp.float32),
        mesh=mesh,
        scratch_types=[
            pltpu.VMEM((rows_per_w, CH), jnp.int32),
            pltpu.VMEM((NB, CH, D), jnp.float32),
            pltpu.VMEM_SHARED((V, D), jnp.float32),
            pltpu.SemaphoreType.DMA((NB,)),
            pltpu.SemaphoreType.DMA((NB,)),
        ],
    )
    def emb_kernel(x_hbm, tab_hbm, out_hbm, idx_v, rows_v, tab_sh, gsem, wsem):
        wid = lax.axis_index("s") * NC + lax.axis_index("c")
        row0 = wid * rows_per_w

        @pl.when(lax.axis_index("s") == 0)
        def _():
            pltpu.sync_copy(tab_hbm, tab_sh)

        pltpu.sync_copy(x_hbm.at[pl.ds(row0, rows_per_w)], idx_v)
        plsc.subcore_barrier()

        def gather(g, s):
            return pltpu.make_async_copy(
                tab_sh.at[idx_v.at[g]], rows_v.at[s], gsem.at[s]
            )

        def write(g, s):
            return pltpu.make_async_copy(
                rows_v.at[s], out_hbm.at[pl.ds((row0 + g) * CH, CH)], wsem.at[s]
            )

        for k in range(LA):
            gather(k, k).start()

        def body(i, carry):
            for s in range(NB):
                g = NB * i + s
                gn = g + LA
                sn = (s + LA) % NB

                @pl.when(gn < rows_per_w)
                def _():
                    @pl.when(g >= NB - LA)
                    def _():
                        write(g, sn).wait()

                    gather(gn, sn).start()

                gather(g, s).wait()
                write(g, s).start()
            return carry

        lax.fori_loop(0, rows_per_w // NB, body, 0)
        for s in range(NB):
            write(0, s).wait()

    return emb_kernel(x2, embedding_list)
